# baseline (speedup 1.0000x reference)
import jax
import jax.numpy as jnp
import numpy as np
from jax.experimental import pallas as pl
from jax.experimental.pallas import tpu as pltpu

_B, _CIN, _H, _W = 4, 3, 512, 512
_S = 16
_CO = 128
_FH, _FW = _H // _S, _W // _S
_K = _CIN * _S * _S


def _patch_conv_kernel(x_ref, e_ref, w_ref, b_ref, o_ref):
    xb = x_ref[0].reshape(_CIN * _H, _W)
    z = jnp.dot(xb, e_ref[...], preferred_element_type=jnp.float32)
    z4 = z.reshape(_CIN, _FH, _S, _W)
    v = jnp.stack([z4[:, :, :, kw * _FW:(kw + 1) * _FW] for kw in range(_S)],
                  axis=3)
    w = w_ref[...]
    b = b_ref[...]
    for h in range(_FH):
        zh = v[:, h].reshape(_K, _FW)
        acc = jnp.dot(w, zh, preferred_element_type=jnp.float32)
        o_ref[0, :, h, :] = jnp.maximum(acc + b, 0.0)


def kernel(x, gts, Wc, bc):
    del gts
    col = np.arange(_W)
    dst = (col % _S) * _FW + col // _S
    em = jnp.asarray((dst[:, None] == np.arange(_W)[None, :]),
                     dtype=jnp.float32)
    wm = Wc.reshape(_CO, _K)
    bm = bc.reshape(_CO, 1)
    out = pl.pallas_call(
        _patch_conv_kernel,
        grid=(_B,),
        in_specs=[
            pl.BlockSpec((1, _CIN, _H, _W), lambda b: (b, 0, 0, 0)),
            pl.BlockSpec((_W, _W), lambda b: (0, 0)),
            pl.BlockSpec((_CO, _K), lambda b: (0, 0)),
            pl.BlockSpec((_CO, 1), lambda b: (0, 0)),
        ],
        out_specs=pl.BlockSpec((1, _CO, _FH, _FW), lambda b: (b, 0, 0, 0)),
        out_shape=jax.ShapeDtypeStruct((_B, _CO, _FH, _FW), jnp.float32),
        compiler_params=pltpu.CompilerParams(
            dimension_semantics=("parallel",)),
    )(x, em, wm, bm)
    return out


# speedup vs baseline: 2.5112x; 2.5112x over previous
import jax
import jax.numpy as jnp
import numpy as np
from jax.experimental import pallas as pl
from jax.experimental.pallas import tpu as pltpu

_B, _CIN, _H, _W = 4, 3, 512, 512
_S = 16
_CO = 128
_FH, _FW = _H // _S, _W // _S
_K = _CIN * _S * _S


def _patch_conv_kernel(x_ref, e_ref, w_ref, b_ref, o_ref):
    xb = x_ref[0].reshape(_CIN * _H, _W)
    z = jnp.dot(xb, e_ref[...], preferred_element_type=jnp.float32)
    z4 = z.reshape(_CIN, _FH, _S, _W)
    v = jnp.stack([z4[:, :, :, kw * _FW:(kw + 1) * _FW] for kw in range(_S)])
    w = w_ref[...]
    b = b_ref[...]
    for h in range(_FH):
        zh = v[:, :, h].reshape(_K, _FW)
        acc = jnp.dot(w, zh, preferred_element_type=jnp.float32)
        o_ref[0, :, h, :] = jnp.maximum(acc + b, 0.0)


def kernel(x, gts, Wc, bc):
    del gts
    col = np.arange(_W)
    dst = (col % _S) * _FW + col // _S
    em = jnp.asarray((dst[:, None] == np.arange(_W)[None, :]),
                     dtype=jnp.float32)
    wm = jnp.transpose(Wc, (0, 3, 1, 2)).reshape(_CO, _K)
    bm = bc.reshape(_CO, 1)
    out = pl.pallas_call(
        _patch_conv_kernel,
        grid=(_B,),
        in_specs=[
            pl.BlockSpec((1, _CIN, _H, _W), lambda b: (b, 0, 0, 0)),
            pl.BlockSpec((_W, _W), lambda b: (0, 0)),
            pl.BlockSpec((_CO, _K), lambda b: (0, 0)),
            pl.BlockSpec((_CO, 1), lambda b: (0, 0)),
        ],
        out_specs=pl.BlockSpec((1, _CO, _FH, _FW), lambda b: (b, 0, 0, 0)),
        out_shape=jax.ShapeDtypeStruct((_B, _CO, _FH, _FW), jnp.float32),
        compiler_params=pltpu.CompilerParams(
            dimension_semantics=("parallel",)),
    )(x, em, wm, bm)
    return out


# speedup vs baseline: 2.5141x; 1.0011x over previous
import jax
import jax.numpy as jnp
import numpy as np
from jax.experimental import pallas as pl
from jax.experimental.pallas import tpu as pltpu

_B, _CIN, _H, _W = 4, 3, 512, 512
_S = 16
_CO = 128
_FH, _FW = _H // _S, _W // _S
_K = _CIN * _S * _S


def _patch_conv_kernel(x_ref, e_ref, w_ref, b_ref, o_ref):
    xb = x_ref[...].reshape(_CIN * _H, _W)
    z = jnp.dot(xb, e_ref[...], preferred_element_type=jnp.float32)
    z4 = z.reshape(_CIN, _FH, _S, _W)
    v = jnp.stack([z4[:, :, :, kw * _FW:(kw + 1) * _FW] for kw in range(_S)])
    w = w_ref[...]
    b = b_ref[...]
    for h in range(_FH):
        zh = v[:, :, h].reshape(_K, _FW)
        acc = jnp.dot(w, zh, preferred_element_type=jnp.float32)
        o_ref[0, :, h, :] = jnp.maximum(acc + b, 0.0)


def kernel(x, gts, Wc, bc):
    del gts
    col = np.arange(_W)
    dst = (col % _S) * _FW + col // _S
    em = jnp.asarray((dst[:, None] == np.arange(_W)[None, :]),
                     dtype=jnp.float32)
    wm = jnp.transpose(Wc, (0, 3, 1, 2)).reshape(_CO, _K)
    bm = bc.reshape(_CO, 1)
    x2 = x.reshape(_B * _CIN * _H, _W)
    out = pl.pallas_call(
        _patch_conv_kernel,
        grid=(_B,),
        in_specs=[
            pl.BlockSpec((_CIN * _H, _W), lambda b: (b, 0)),
            pl.BlockSpec((_W, _W), lambda b: (0, 0)),
            pl.BlockSpec((_CO, _K), lambda b: (0, 0)),
            pl.BlockSpec((_CO, 1), lambda b: (0, 0)),
        ],
        out_specs=pl.BlockSpec((1, _CO, _FH, _FW), lambda b: (b, 0, 0, 0)),
        out_shape=jax.ShapeDtypeStruct((_B, _CO, _FH, _FW), jnp.float32),
        compiler_params=pltpu.CompilerParams(
            dimension_semantics=("parallel",)),
    )(x2, em, wm, bm)
    return out


# speedup vs baseline: 2.5447x; 1.0122x over previous
import jax
import jax.numpy as jnp
import numpy as np
from jax.experimental import pallas as pl
from jax.experimental.pallas import tpu as pltpu

_B, _CIN, _H, _W = 4, 3, 512, 512
_S = 16
_CO = 128
_FH, _FW = _H // _S, _W // _S
_K = _CIN * _S * _S
_HB = 16


def _patch_conv_kernel(x_ref, e_ref, w_ref, b_ref, o_ref):
    xb = x_ref[...].reshape(_CIN * _HB * _S, _W)
    z = jnp.dot(xb, e_ref[...], preferred_element_type=jnp.float32)
    z4 = z.reshape(_CIN, _HB, _S, _W)
    v = jnp.stack([z4[:, :, :, kw * _FW:(kw + 1) * _FW] for kw in range(_S)])
    w = w_ref[...]
    b = b_ref[...]
    for h in range(_HB):
        zh = v[:, :, h].reshape(_K, _FW)
        acc = jnp.dot(w, zh, preferred_element_type=jnp.float32)
        o_ref[0, :, h, :] = jnp.maximum(acc + b, 0.0)


def kernel(x, gts, Wc, bc):
    del gts
    col = np.arange(_W)
    dst = (col % _S) * _FW + col // _S
    em = jnp.asarray((dst[:, None] == np.arange(_W)[None, :]),
                     dtype=jnp.float32)
    wm = jnp.transpose(Wc, (0, 3, 1, 2)).reshape(_CO, _K)
    bm = bc.reshape(_CO, 1)
    xs = x.reshape(_B, _CIN, _FH // _HB, _HB * _S, _W)
    out = pl.pallas_call(
        _patch_conv_kernel,
        grid=(_B, _FH // _HB),
        in_specs=[
            pl.BlockSpec((1, _CIN, 1, _HB * _S, _W),
                         lambda b, h: (b, 0, h, 0, 0)),
            pl.BlockSpec((_W, _W), lambda b, h: (0, 0)),
            pl.BlockSpec((_CO, _K), lambda b, h: (0, 0)),
            pl.BlockSpec((_CO, 1), lambda b, h: (0, 0)),
        ],
        out_specs=pl.BlockSpec((1, _CO, _HB, _FW), lambda b, h: (b, 0, h, 0)),
        out_shape=jax.ShapeDtypeStruct((_B, _CO, _FH, _FW), jnp.float32),
        compiler_params=pltpu.CompilerParams(
            dimension_semantics=("parallel", "parallel")),
    )(xs, em, wm, bm)
    return out
